# baseline (speedup 1.0000x reference)
import jax
import jax.numpy as jnp
from jax import lax
from jax.experimental import pallas as pl
from jax.experimental.pallas import tpu as pltpu
from jax.experimental.pallas import tpu_sc as plsc

_D = 256
_TMP = 1e-05
_MAX_ITER = 100
_L = 16
_NCHUNK = _D // _L


def _cmp_exchange(lo, hi):
    (ak, av), (bk, bv) = lo, hi
    m = ak <= bk
    return (
        (jnp.where(m, ak, bk), jnp.where(m, av, bv)),
        (jnp.where(m, bk, ak), jnp.where(m, bv, av)),
    )


def _bitonic_merge(run_a, run_b):
    n = len(run_b)
    b_rev = [(lax.rev(k, (0,)), lax.rev(v, (0,))) for (k, v) in reversed(run_b)]
    arr = list(run_a) + b_rev
    stride = n
    while stride >= 1:
        for base in range(0, len(arr), 2 * stride):
            for i in range(base, base + stride):
                arr[i], arr[i + stride] = _cmp_exchange(arr[i], arr[i + stride])
        stride //= 2
    return [plsc.sort_key_val(k, v) for (k, v) in arr]


def _sort256(kv):
    runs = [[plsc.sort_key_val(k, v)] for (k, v) in kv]
    while len(runs) > 1:
        runs = [
            _bitonic_merge(runs[i], runs[i + 1]) for i in range(0, len(runs), 2)
        ]
    return runs[0]


def _fw_sc(theta_hbm, out_hbm, t_v, mu_v, s_v):
    tile0 = (lax.axis_index("c") == 0) & (lax.axis_index("s") == 0)

    @pl.when(tile0)
    def _():
        pltpu.sync_copy(theta_hbm, t_v)
        iota_i = lax.iota(jnp.int32, _L)
        iota_f = iota_i.astype(jnp.float32)
        for c in range(_NCHUNK):
            ds = pl.ds(c * _L, _L)
            t_v[ds] = t_v[ds] / jnp.float32(_TMP)

        def lmo_into(dst_ref, g_chunks):
            kv = [(g_chunks[c], iota_i + c * _L) for c in range(_NCHUNK)]
            sorted_kv = _sort256(kv)
            for c in range(_NCHUNK):
                _, idx = sorted_kv[c]
                sval = jnp.float32(_D - c * _L) - iota_f
                plsc.store_scatter(dst_ref, [idx], sval)

        lmo_into(mu_v, [-t_v[pl.ds(c * _L, _L)] for c in range(_NCHUNK)])

        def cond(carry):
            it, done = carry
            return (it < _MAX_ITER) & jnp.logical_not(done)

        def body(carry):
            it, _ = carry
            lmo_into(
                s_v,
                [
                    mu_v[pl.ds(c * _L, _L)] - t_v[pl.ds(c * _L, _L)]
                    for c in range(_NCHUNK)
                ],
            )
            num_acc = jnp.zeros((_L,), jnp.float32)
            den_acc = jnp.zeros((_L,), jnp.float32)
            for c in range(_NCHUNK):
                ds = pl.ds(c * _L, _L)
                g = mu_v[ds] - t_v[ds]
                dv = s_v[ds] - mu_v[ds]
                num_acc = num_acc + g * dv
                den_acc = den_acc + dv * dv
            num = -jnp.sum(num_acc)
            den = jnp.sum(den_acc)
            num_v = jnp.full((_L,), num, jnp.float32)
            den_v = jnp.full((_L,), den, jnp.float32)
            gamma_v = jnp.where(
                den_v > 0,
                jnp.clip(num_v / den_v, 0.0, 1.0),
                jnp.zeros((_L,), jnp.float32),
            )
            for c in range(_NCHUNK):
                ds = pl.ds(c * _L, _L)
                mu_v[ds] = mu_v[ds] + gamma_v * (s_v[ds] - mu_v[ds])
            return it + jnp.int32(1), num <= 0.0

        lax.while_loop(cond, body, (jnp.int32(0), jnp.bool_(False)))
        pltpu.sync_copy(mu_v, out_hbm)


def kernel(theta):
    t_in = theta.astype(jnp.float32).reshape(_D)
    out = pl.kernel(
        _fw_sc,
        out_type=jax.ShapeDtypeStruct((_D,), jnp.float32),
        mesh=plsc.VectorSubcoreMesh(
            core_axis_name="c", subcore_axis_name="s", num_cores=1, num_subcores=1
        ),
        scratch_types=[pltpu.VMEM((_D,), jnp.float32) for _ in range(3)],
        compiler_params=pltpu.CompilerParams(needs_layout_passes=False),
    )(t_in)
    return out
